# Initial kernel scaffold; baseline (speedup 1.0000x reference)
import jax
import jax.numpy as jnp
from jax.experimental import pallas as pl


def kernel(x, y, W, c_w):
    raise NotImplementedError("write your pallas kernel here")


# speedup vs baseline: 1.3018x; 1.3018x over previous
import functools

import jax
import jax.numpy as jnp
from jax import lax
from jax.experimental import pallas as pl
from jax.experimental.pallas import tpu as pltpu
from jax.experimental.pallas import tpu_sc as plsc

B, D, C = 16384, 512, 512


_BB = 512


def _winner_body(x_ref, w_ref, out_ref):
    x = x_ref[...]
    w = w_ref[...]
    xw = lax.dot_general(
        x, w, (((1,), (1,)), ((), ())),
        preferred_element_type=jnp.float32,
    )
    x2 = jnp.sum(x * x, axis=1, keepdims=True)
    w2 = jnp.sum(w * w, axis=1)[None, :]
    d2 = jnp.maximum(x2 + w2 - 2.0 * xw, 1e-12)
    out_ref[...] = jnp.argmin(d2, axis=1).astype(jnp.int32)


def _winner_call(x, W):
    grid = B // _BB
    return pl.pallas_call(
        _winner_body,
        grid=(grid,),
        in_specs=[
            pl.BlockSpec((_BB, D), lambda i: (i, 0)),
            pl.BlockSpec((C, D), lambda i: (0, 0)),
        ],
        out_specs=pl.BlockSpec((_BB,), lambda i: (i,)),
        out_shape=jax.ShapeDtypeStruct((B,), jnp.int32),
    )(x, W)


_info = plsc.get_sparse_core_info()
_NC, _NS = _info.num_cores, _info.num_subcores
_NW = _NC * _NS
_BPW = B // _NW
_CHUNK = 64
_NCHUNK = _BPW // _CHUNK


def _gather_body(cw_hbm, idx_hbm, out_hbm, idx_v, rows_v, gsem, wsem0, wsem1):
    wid = lax.axis_index("s") * _NC + lax.axis_index("c")
    base = wid * _BPW
    for j in range(_NCHUNK):
        pltpu.sync_copy(idx_hbm.at[pl.ds(base + j * _CHUNK, _CHUNK)], idx_v.at[j])
    wsems = (wsem0, wsem1)

    def write_copy(j):
        return pltpu.make_async_copy(
            rows_v.at[j % 2],
            out_hbm.at[pl.ds(base + j * _CHUNK, _CHUNK)],
            wsems[j % 2])

    for j in range(_NCHUNK):
        buf = j % 2
        if j >= 2:
            write_copy(j - 2).wait()
        pltpu.async_copy(cw_hbm.at[idx_v.at[j]], rows_v.at[buf], gsem).wait()
        write_copy(j).start()
    for j in range(max(_NCHUNK - 2, 0), _NCHUNK):
        write_copy(j).wait()


def _gather_call(c_w, winner):
    mesh = plsc.VectorSubcoreMesh(core_axis_name="c", subcore_axis_name="s")
    k = functools.partial(
        pl.kernel,
        mesh=mesh,
        out_type=jax.ShapeDtypeStruct((B, D), jnp.int32),
        scratch_types=[
            pltpu.VMEM((_NCHUNK, _CHUNK), jnp.int32),
            pltpu.VMEM((2, _CHUNK, D), jnp.int32),
            pltpu.SemaphoreType.DMA,
            pltpu.SemaphoreType.DMA,
            pltpu.SemaphoreType.DMA,
        ],
    )(_gather_body)
    return k(c_w, winner)


def kernel(x, y, W, c_w):
    winner = _winner_call(x, W)
    return _gather_call(c_w, winner)
